# Initial kernel scaffold; baseline (speedup 1.0000x reference)
import jax
import jax.numpy as jnp
from jax.experimental import pallas as pl


def kernel(x, edge_index, batch, W1, b1, W2, b2):
    raise NotImplementedError("write your pallas kernel here")


# speedup vs baseline: 13.2649x; 13.2649x over previous
import functools

import jax
import jax.numpy as jnp
from jax import lax
from jax.experimental import pallas as pl
from jax.experimental.pallas import tpu as pltpu
from jax.experimental.pallas import tpu_sc as plsc

N = 10000
N2 = 10240
E = 320000
D = 128
G = 64

NC = 2
NS = 16
NW = NC * NS
PER_W = E // NW
CH = 80
NCHUNK = PER_W // CH
RPS = N2 // NS
ZR = 128

_mesh = plsc.VectorSubcoreMesh(core_axis_name="c", subcore_axis_name="s")


def _deg_body(dst_h, zeros_h, out_h, idxd_v, ones_v, acc_sh, dummy_sem):
    c = lax.axis_index("c")
    s = lax.axis_index("s")
    wid = s * NC + c

    def fill_ones(i, _):
        ones_v[i, :] = jnp.ones((16,), jnp.float32)
        return 0

    lax.fori_loop(0, CH, fill_ones, 0)
    pltpu.sync_copy(zeros_h, acc_sh.at[pl.ds(s * RPS, RPS)])
    plsc.subcore_barrier()

    base = wid * PER_W

    def chunk(j, _):
        off = base + j * CH
        pltpu.sync_copy(dst_h.at[pl.ds(off, CH)], idxd_v.at[0])
        pltpu.sync_copy(ones_v, acc_sh.at[idxd_v.at[0]], add=True)
        return 0

    lax.fori_loop(0, NCHUNK, chunk, 0)
    plsc.subcore_barrier()
    pltpu.sync_copy(acc_sh.at[pl.ds(s * RPS, RPS)],
                    out_h.at[c, pl.ds(s * RPS, RPS)])


@jax.jit
def _deg_call(dst, zeros16):
    f = functools.partial(
        pl.kernel,
        out_type=jax.ShapeDtypeStruct((NC, N2, 16), jnp.float32),
        mesh=_mesh,
        scratch_types=[
            pltpu.VMEM((2, CH), jnp.int32),
            pltpu.VMEM((CH, 16), jnp.float32),
            pltpu.VMEM_SHARED((N2, 16), jnp.float32),
            pltpu.SemaphoreType.DMA,
        ],
    )(_deg_body)
    return f(dst, zeros16)


def _prop_body(src_h, dst_h, tab_h, zeros_h, out_h, idxs_v, idxd_v, rows_v,
               acc_sh, sem):
    c = lax.axis_index("c")
    s = lax.axis_index("s")
    wid = s * NC + c

    for k in range(RPS // ZR):
        pltpu.sync_copy(zeros_h, acc_sh.at[pl.ds(s * RPS + k * ZR, ZR)])
    plsc.subcore_barrier()

    base = wid * PER_W

    def chunk(j, _):
        off = base + j * CH
        pltpu.sync_copy(src_h.at[pl.ds(off, CH)], idxs_v)
        pltpu.sync_copy(dst_h.at[pl.ds(off, CH)], idxd_v.at[0])
        pltpu.async_copy(tab_h.at[idxs_v], rows_v, sem).wait()
        pltpu.sync_copy(rows_v, acc_sh.at[idxd_v.at[0]], add=True)
        return 0

    lax.fori_loop(0, NCHUNK, chunk, 0)
    plsc.subcore_barrier()
    pltpu.sync_copy(acc_sh.at[pl.ds(s * RPS, RPS)],
                    out_h.at[c, pl.ds(s * RPS, RPS)])


@jax.jit
def _prop_call(src, dst, table, zeros):
    f = functools.partial(
        pl.kernel,
        out_type=jax.ShapeDtypeStruct((NC, N2, D), jnp.float32),
        mesh=_mesh,
        scratch_types=[
            pltpu.VMEM((CH,), jnp.int32),
            pltpu.VMEM((2, CH), jnp.int32),
            pltpu.VMEM((CH, D), jnp.float32),
            pltpu.VMEM_SHARED((N2, D), jnp.float32),
            pltpu.SemaphoreType.DMA,
        ],
    )(_prop_body)
    return f(src, dst, table, zeros)


R_BLK = 2048
N_BLK = N2 // R_BLK


def _dinv_from_degp(degp_blk):
    deg = degp_blk[0, :, 0] + degp_blk[1, :, 0] + 1.0
    return lax.rsqrt(deg)


def _scale_body(x_ref, w_ref, degp_ref, out_ref):
    h = jnp.dot(x_ref[...], w_ref[...], preferred_element_type=jnp.float32)
    dinv = _dinv_from_degp(degp_ref[...])
    out_ref[...] = h * dinv[:, None]


@jax.jit
def _scale_call(x, W1, degP):
    return pl.pallas_call(
        _scale_body,
        grid=(N_BLK,),
        in_specs=[
            pl.BlockSpec((R_BLK, D), lambda i: (i, 0)),
            pl.BlockSpec((D, D), lambda i: (0, 0)),
            pl.BlockSpec((NC, R_BLK, 16), lambda i: (0, i, 0)),
        ],
        out_specs=pl.BlockSpec((R_BLK, D), lambda i: (i, 0)),
        out_shape=jax.ShapeDtypeStruct((N2, D), jnp.float32),
    )(x, W1, degP)


def _relu_body(p_ref, hp_ref, degp_ref, b1_ref, out_ref):
    dinv = _dinv_from_degp(degp_ref[...])
    agg = dinv[:, None] * (p_ref[0] + p_ref[1] + hp_ref[...]) + b1_ref[...]
    out_ref[...] = jnp.maximum(agg, 0.0) * dinv[:, None]


@jax.jit
def _relu_call(P, h1p, degP, b1):
    return pl.pallas_call(
        _relu_body,
        grid=(N_BLK,),
        in_specs=[
            pl.BlockSpec((NC, R_BLK, D), lambda i: (0, i, 0)),
            pl.BlockSpec((R_BLK, D), lambda i: (i, 0)),
            pl.BlockSpec((NC, R_BLK, 16), lambda i: (0, i, 0)),
            pl.BlockSpec((1, D), lambda i: (0, 0)),
        ],
        out_specs=pl.BlockSpec((R_BLK, D), lambda i: (i, 0)),
        out_shape=jax.ShapeDtypeStruct((N2, D), jnp.float32),
    )(P, h1p, degP, b1.reshape(1, D))


def _pool_body(q_ref, hp_ref, degp_ref, batch_ref, w2_ref, b2_ref, out_ref,
               acc_s, cnt_s):
    i = pl.program_id(0)

    @pl.when(i == 0)
    def _():
        acc_s[...] = jnp.zeros((G, D), jnp.float32)
        cnt_s[...] = jnp.zeros((G, D), jnp.float32)

    dinv = _dinv_from_degp(degp_ref[...])
    u = dinv[:, None] * (q_ref[0] + q_ref[1] + hp_ref[...])
    b = batch_ref[0, 0, :]
    onehot_t = (b[None, :] == lax.broadcasted_iota(jnp.int32, (G, R_BLK), 0))
    onehot_t = onehot_t.astype(jnp.float32)
    acc_s[...] += jnp.dot(onehot_t, u, preferred_element_type=jnp.float32)
    cnt_s[...] += jnp.dot(onehot_t, jnp.ones((R_BLK, D), jnp.float32),
                          preferred_element_type=jnp.float32)

    @pl.when(i == N_BLK - 1)
    def _():
        mean = acc_s[...] / jnp.maximum(cnt_s[...], 1.0)
        out_ref[...] = (jnp.dot(mean, w2_ref[...],
                                preferred_element_type=jnp.float32)
                        + b2_ref[...])


@jax.jit
def _pool_call(Q, h2p, degP, batch3, W2, b2):
    return pl.pallas_call(
        _pool_body,
        grid=(N_BLK,),
        in_specs=[
            pl.BlockSpec((NC, R_BLK, D), lambda i: (0, i, 0)),
            pl.BlockSpec((R_BLK, D), lambda i: (i, 0)),
            pl.BlockSpec((NC, R_BLK, 16), lambda i: (0, i, 0)),
            pl.BlockSpec((1, 1, R_BLK), lambda i: (i, 0, 0)),
            pl.BlockSpec((D, D), lambda i: (0, 0)),
            pl.BlockSpec((1, D), lambda i: (0, 0)),
        ],
        out_specs=pl.BlockSpec((G, D), lambda i: (0, 0)),
        out_shape=jax.ShapeDtypeStruct((G, D), jnp.float32),
        scratch_shapes=[
            pltpu.VMEM((G, D), jnp.float32),
            pltpu.VMEM((G, D), jnp.float32),
        ],
    )(Q, h2p, degP, batch3, W2, b2.reshape(1, D))


def kernel(x, edge_index, batch, W1, b1, W2, b2):
    zeros = jnp.zeros((ZR, D), jnp.float32)
    zeros16 = jnp.zeros((RPS, 16), jnp.float32)
    src = edge_index[0]
    dst = edge_index[1]
    x_pad = jnp.pad(x, ((0, N2 - N), (0, 0)))
    batch_pad = jnp.pad(batch, (0, N2 - N), constant_values=G)
    degP = _deg_call(dst, zeros16)
    h1p = _scale_call(x_pad, W1, degP)
    P = _prop_call(src, dst, h1p, zeros)
    h2p = _relu_call(P, h1p, degP, b1)
    Q = _prop_call(src, dst, h2p, zeros)
    batch3 = batch_pad.reshape(N_BLK, 1, R_BLK)
    return _pool_call(Q, h2p, degP, batch3, W2, b2)


# speedup vs baseline: 17.9211x; 1.3510x over previous
import functools

import jax
import jax.numpy as jnp
from jax import lax
from jax.experimental import pallas as pl
from jax.experimental.pallas import tpu as pltpu
from jax.experimental.pallas import tpu_sc as plsc

N = 10000
N2 = 10240
E = 320000
D = 128
G = 64

NC = 2
NS = 16
NW = NC * NS
PER_W = E // NW
CH = 80
NCHUNK = PER_W // CH
SUB = 1
CHG = CH * SUB
NCG = PER_W // CHG
RPS = N2 // NS
ZR = 128

_mesh = plsc.VectorSubcoreMesh(core_axis_name="c", subcore_axis_name="s")


NRING = 3
NT = (NCG - 5) // NRING


def _deg_body(dst_h, zeros_h, out_h, id0, id1, id2, ones_v, acc_sh,
              si0, si1, si2, ss0, ss1, ss2):
    c = lax.axis_index("c")
    s = lax.axis_index("s")
    wid = s * NC + c
    base = wid * PER_W
    sem_i = (si0, si1, si2)
    sem_s = (ss0, ss1, ss2)
    idxd = (id0, id1, id2)

    def fill_ones(i, _):
        ones_v[i, :] = jnp.ones((16,), jnp.float32)
        return 0

    lax.fori_loop(0, CH, fill_ones, 0)
    pltpu.async_copy(zeros_h, acc_sh.at[pl.ds(s * RPS, RPS)], ss0)
    pltpu.make_async_copy(zeros_h, acc_sh.at[pl.ds(s * RPS, RPS)], ss0).wait()
    plsc.subcore_barrier()

    def pre(j, b):
        pltpu.async_copy(dst_h.at[pl.ds(base + j * CH, CH)], idxd[b].at[0],
                         sem_i[b])

    def op(j, b):
        pltpu.make_async_copy(dst_h.at[pl.ds(base, CH)], idxd[b].at[0],
                              sem_i[b]).wait()
        pltpu.sync_copy(ones_v, acc_sh.at[idxd[b].at[0]], add=True)

    def scd(b):
        pass

    for b in range(NRING):
        pre(b, b)

    def body(t, _):
        j = NRING * t
        for b in range(NRING):
            op(j + b, b)
        for b in range(NRING):
            scd(b)
            pre(j + NRING + b, b)
        return 0

    lax.fori_loop(0, NT, body, 0)
    j = NRING * NT
    op(j, 0)
    op(j + 1, 1)
    op(j + 2, 2)
    scd(0)
    pre(j + 3, 0)
    scd(1)
    pre(j + 4, 1)
    op(j + 3, 0)
    op(j + 4, 1)
    scd(2)
    scd(0)
    scd(1)
    plsc.subcore_barrier()
    pltpu.sync_copy(acc_sh.at[pl.ds(s * RPS, RPS)],
                    out_h.at[c, pl.ds(s * RPS, RPS)])


@jax.jit
def _deg_call(dst, zeros16):
    f = functools.partial(
        pl.kernel,
        out_type=jax.ShapeDtypeStruct((NC, N2, 16), jnp.float32),
        mesh=_mesh,
        scratch_types=[
            pltpu.VMEM((1, CH), jnp.int32),
            pltpu.VMEM((1, CH), jnp.int32),
            pltpu.VMEM((1, CH), jnp.int32),
            pltpu.VMEM((CH, 16), jnp.float32),
            pltpu.VMEM_SHARED((N2, 16), jnp.float32),
            pltpu.SemaphoreType.DMA,
            pltpu.SemaphoreType.DMA,
            pltpu.SemaphoreType.DMA,
            pltpu.SemaphoreType.DMA,
            pltpu.SemaphoreType.DMA,
            pltpu.SemaphoreType.DMA,
        ],
    )(_deg_body)
    return f(dst, zeros16)


def _prop_body(src_h, dst_h, tab_h, zeros_h, out_h, is0, is1, is2,
               id0, id1, id2, rows_v, acc_sh,
               sg0, sg1, sg2, si0, si1, si2, ss0, ss1, ss2):
    c = lax.axis_index("c")
    s = lax.axis_index("s")
    wid = s * NC + c
    base = wid * PER_W
    sem_g = (sg0, sg1, sg2)
    sem_i = (si0, si1, si2)
    sem_s = (ss0, ss1, ss2)
    idxs = (is0, is1, is2)
    idxd = (id0, id1, id2)

    for k in range(RPS // ZR):
        pltpu.async_copy(zeros_h, acc_sh.at[pl.ds(s * RPS + k * ZR, ZR)], ss0)
    for k in range(RPS // ZR):
        pltpu.make_async_copy(zeros_h, acc_sh.at[pl.ds(s * RPS, ZR)],
                              ss0).wait()
    plsc.subcore_barrier()

    def pre(j, b):
        pltpu.async_copy(dst_h.at[pl.ds(base + j * CH, CH)], idxd[b].at[0],
                         sem_i[b])
        pltpu.async_copy(src_h.at[pl.ds(base + j * CH, CH)], idxs[b].at[0],
                         sem_i[b])

    def op(j, b):
        pltpu.make_async_copy(dst_h.at[pl.ds(base, CH)], idxd[b].at[0],
                              sem_i[b]).wait()
        pltpu.make_async_copy(dst_h.at[pl.ds(base, CH)], idxs[b].at[0],
                              sem_i[b]).wait()
        pltpu.sync_copy(tab_h.at[idxs[b].at[0]], rows_v.at[b])
        pltpu.sync_copy(rows_v.at[b], acc_sh.at[idxd[b].at[0]], add=True)

    def scd(b):
        pass

    for b in range(NRING):
        pre(b, b)

    def body(t, _):
        j = NRING * t
        for b in range(NRING):
            op(j + b, b)
        for b in range(NRING):
            scd(b)
            pre(j + NRING + b, b)
        return 0

    lax.fori_loop(0, NT, body, 0)
    j = NRING * NT
    op(j, 0)
    op(j + 1, 1)
    op(j + 2, 2)
    scd(0)
    pre(j + 3, 0)
    scd(1)
    pre(j + 4, 1)
    op(j + 3, 0)
    op(j + 4, 1)
    scd(2)
    scd(0)
    scd(1)
    plsc.subcore_barrier()
    pltpu.sync_copy(acc_sh.at[pl.ds(s * RPS, RPS)],
                    out_h.at[c, pl.ds(s * RPS, RPS)])


@jax.jit
def _prop_call(src, dst, table, zeros):
    f = functools.partial(
        pl.kernel,
        out_type=jax.ShapeDtypeStruct((NC, N2, D), jnp.float32),
        mesh=_mesh,
        scratch_types=[
            pltpu.VMEM((1, CH), jnp.int32),
            pltpu.VMEM((1, CH), jnp.int32),
            pltpu.VMEM((1, CH), jnp.int32),
            pltpu.VMEM((1, CH), jnp.int32),
            pltpu.VMEM((1, CH), jnp.int32),
            pltpu.VMEM((1, CH), jnp.int32),
            pltpu.VMEM((NRING, CH, D), jnp.float32),
            pltpu.VMEM_SHARED((N2, D), jnp.float32),
        ] + [pltpu.SemaphoreType.DMA] * 9,
    )(_prop_body)
    return f(src, dst, table, zeros)


R_BLK = 2048
N_BLK = N2 // R_BLK


def _dinv_from_degp(degp_blk):
    deg = degp_blk[0, :, 0] + degp_blk[1, :, 0] + 1.0
    return lax.rsqrt(deg)


def _scale_body(x_ref, w_ref, degp_ref, out_ref):
    h = jnp.dot(x_ref[...], w_ref[...], preferred_element_type=jnp.float32)
    dinv = _dinv_from_degp(degp_ref[...])
    out_ref[...] = h * dinv[:, None]


@jax.jit
def _scale_call(x, W1, degP):
    return pl.pallas_call(
        _scale_body,
        grid=(N_BLK,),
        in_specs=[
            pl.BlockSpec((R_BLK, D), lambda i: (i, 0)),
            pl.BlockSpec((D, D), lambda i: (0, 0)),
            pl.BlockSpec((NC, R_BLK, 16), lambda i: (0, i, 0)),
        ],
        out_specs=pl.BlockSpec((R_BLK, D), lambda i: (i, 0)),
        out_shape=jax.ShapeDtypeStruct((N2, D), jnp.float32),
    )(x, W1, degP)


def _relu_body(p_ref, hp_ref, degp_ref, b1_ref, out_ref):
    dinv = _dinv_from_degp(degp_ref[...])
    agg = dinv[:, None] * (p_ref[0] + p_ref[1] + hp_ref[...]) + b1_ref[...]
    out_ref[...] = jnp.maximum(agg, 0.0) * dinv[:, None]


@jax.jit
def _relu_call(P, h1p, degP, b1):
    return pl.pallas_call(
        _relu_body,
        grid=(N_BLK,),
        in_specs=[
            pl.BlockSpec((NC, R_BLK, D), lambda i: (0, i, 0)),
            pl.BlockSpec((R_BLK, D), lambda i: (i, 0)),
            pl.BlockSpec((NC, R_BLK, 16), lambda i: (0, i, 0)),
            pl.BlockSpec((1, D), lambda i: (0, 0)),
        ],
        out_specs=pl.BlockSpec((R_BLK, D), lambda i: (i, 0)),
        out_shape=jax.ShapeDtypeStruct((N2, D), jnp.float32),
    )(P, h1p, degP, b1.reshape(1, D))


def _pool_body(q_ref, hp_ref, degp_ref, batch_ref, w2_ref, b2_ref, out_ref,
               acc_s, cnt_s):
    i = pl.program_id(0)

    @pl.when(i == 0)
    def _():
        acc_s[...] = jnp.zeros((G, D), jnp.float32)
        cnt_s[...] = jnp.zeros((G, D), jnp.float32)

    dinv = _dinv_from_degp(degp_ref[...])
    u = dinv[:, None] * (q_ref[0] + q_ref[1] + hp_ref[...])
    b = batch_ref[0, 0, :]
    onehot_t = (b[None, :] == lax.broadcasted_iota(jnp.int32, (G, R_BLK), 0))
    onehot_t = onehot_t.astype(jnp.float32)
    acc_s[...] += jnp.dot(onehot_t, u, preferred_element_type=jnp.float32)
    cnt_s[...] += jnp.dot(onehot_t, jnp.ones((R_BLK, D), jnp.float32),
                          preferred_element_type=jnp.float32)

    @pl.when(i == N_BLK - 1)
    def _():
        mean = acc_s[...] / jnp.maximum(cnt_s[...], 1.0)
        out_ref[...] = (jnp.dot(mean, w2_ref[...],
                                preferred_element_type=jnp.float32)
                        + b2_ref[...])


@jax.jit
def _pool_call(Q, h2p, degP, batch3, W2, b2):
    return pl.pallas_call(
        _pool_body,
        grid=(N_BLK,),
        in_specs=[
            pl.BlockSpec((NC, R_BLK, D), lambda i: (0, i, 0)),
            pl.BlockSpec((R_BLK, D), lambda i: (i, 0)),
            pl.BlockSpec((NC, R_BLK, 16), lambda i: (0, i, 0)),
            pl.BlockSpec((1, 1, R_BLK), lambda i: (i, 0, 0)),
            pl.BlockSpec((D, D), lambda i: (0, 0)),
            pl.BlockSpec((1, D), lambda i: (0, 0)),
        ],
        out_specs=pl.BlockSpec((G, D), lambda i: (0, 0)),
        out_shape=jax.ShapeDtypeStruct((G, D), jnp.float32),
        scratch_shapes=[
            pltpu.VMEM((G, D), jnp.float32),
            pltpu.VMEM((G, D), jnp.float32),
        ],
    )(Q, h2p, degP, batch3, W2, b2.reshape(1, D))


def kernel(x, edge_index, batch, W1, b1, W2, b2):
    zeros = jnp.zeros((ZR, D), jnp.float32)
    zeros16 = jnp.zeros((RPS, 16), jnp.float32)
    src = edge_index[0]
    dst = edge_index[1]
    x_pad = jnp.pad(x, ((0, N2 - N), (0, 0)))
    batch_pad = jnp.pad(batch, (0, N2 - N), constant_values=G)
    degP = _deg_call(dst, zeros16)
    h1p = _scale_call(x_pad, W1, degP)
    P = _prop_call(src, dst, h1p, zeros)
    h2p = _relu_call(P, h1p, degP, b1)
    Q = _prop_call(src, dst, h2p, zeros)
    batch3 = batch_pad.reshape(N_BLK, 1, R_BLK)
    return _pool_call(Q, h2p, degP, batch3, W2, b2)


# speedup vs baseline: 22.6894x; 1.2661x over previous
import functools

import jax
import jax.numpy as jnp
from jax import lax
from jax.experimental import pallas as pl
from jax.experimental.pallas import tpu as pltpu
from jax.experimental.pallas import tpu_sc as plsc

N = 10000
N2 = 10240
E = 320000
D = 128
G = 64

NC = 2
NS = 16
NW = NC * NS
PER_W = E // NW
CH = 80
NCHUNK = PER_W // CH
SUB = 1
CHG = CH * SUB
NCG = PER_W // CHG
RPS = N2 // NS
ZR = 128

_mesh = plsc.VectorSubcoreMesh(core_axis_name="c", subcore_axis_name="s")


NRING = 3
NT = (NCG - 5) // NRING


def _deg_body(dst_h, zeros_h, out_h, id0, id1, id2, ones_v, acc_sh,
              si0, si1, si2, ss0, ss1, ss2):
    c = lax.axis_index("c")
    s = lax.axis_index("s")
    wid = s * NC + c
    base = wid * PER_W
    sem_i = (si0, si1, si2)
    sem_s = (ss0, ss1, ss2)
    idxd = (id0, id1, id2)

    def fill_ones(i, _):
        ones_v[i, :] = jnp.ones((16,), jnp.float32)
        return 0

    lax.fori_loop(0, CH, fill_ones, 0)
    pltpu.async_copy(zeros_h, acc_sh.at[pl.ds(s * RPS, RPS)], ss0)
    pltpu.make_async_copy(zeros_h, acc_sh.at[pl.ds(s * RPS, RPS)], ss0).wait()
    plsc.subcore_barrier()

    def pre(j, b):
        pltpu.async_copy(dst_h.at[pl.ds(base + j * CH, CH)], idxd[b].at[0],
                         sem_i[b])

    def iwait(b):
        pltpu.make_async_copy(dst_h.at[pl.ds(base, CH)], idxd[b].at[0],
                              sem_i[b]).wait()

    def sfire(b):
        return pltpu.async_copy(ones_v, acc_sh.at[idxd[b].at[0]], sem_s[b],
                                add=True)

    for b in range(NRING):
        pre(b, b)
    for b in range(NRING):
        iwait(b)

    def body(t, _):
        j = NRING * t
        sd = [sfire(b) for b in range(NRING)]
        for b in range(NRING):
            sd[b].wait()
            pre(j + NRING + b, b)
        for b in range(NRING):
            iwait(b)
        return 0

    lax.fori_loop(0, NT, body, 0)
    j = NRING * NT
    sd = [sfire(b) for b in range(NRING)]
    sd[0].wait()
    pre(j + 3, 0)
    iwait(0)
    sd[1].wait()
    pre(j + 4, 1)
    iwait(1)
    s0 = sfire(0)
    s1 = sfire(1)
    sd[2].wait()
    s0.wait()
    s1.wait()
    plsc.subcore_barrier()
    pltpu.sync_copy(acc_sh.at[pl.ds(s * RPS, RPS)],
                    out_h.at[c, pl.ds(s * RPS, RPS)])


@jax.jit
def _deg_call(dst, zeros16):
    f = functools.partial(
        pl.kernel,
        out_type=jax.ShapeDtypeStruct((NC, N2, 16), jnp.float32),
        mesh=_mesh,
        scratch_types=[
            pltpu.VMEM((1, CH), jnp.int32),
            pltpu.VMEM((1, CH), jnp.int32),
            pltpu.VMEM((1, CH), jnp.int32),
            pltpu.VMEM((CH, 16), jnp.float32),
            pltpu.VMEM_SHARED((N2, 16), jnp.float32),
            pltpu.SemaphoreType.DMA,
            pltpu.SemaphoreType.DMA,
            pltpu.SemaphoreType.DMA,
            pltpu.SemaphoreType.DMA,
            pltpu.SemaphoreType.DMA,
            pltpu.SemaphoreType.DMA,
        ],
    )(_deg_body)
    return f(dst, zeros16)


def _prop_body(src_h, dst_h, tab_h, zeros_h, out_h, is0, is1, is2,
               id0, id1, id2, rows_v, acc_sh,
               sg0, sg1, sg2, si0, si1, si2, ss0, ss1, ss2):
    c = lax.axis_index("c")
    s = lax.axis_index("s")
    wid = s * NC + c
    base = wid * PER_W
    sem_g = (sg0, sg1, sg2)
    sem_i = (si0, si1, si2)
    sem_s = (ss0, ss1, ss2)
    idxs = (is0, is1, is2)
    idxd = (id0, id1, id2)

    for k in range(RPS // ZR):
        pltpu.async_copy(zeros_h, acc_sh.at[pl.ds(s * RPS + k * ZR, ZR)], ss0)
    for k in range(RPS // ZR):
        pltpu.make_async_copy(zeros_h, acc_sh.at[pl.ds(s * RPS, ZR)],
                              ss0).wait()
    plsc.subcore_barrier()

    def pre(j, b):
        pltpu.async_copy(dst_h.at[pl.ds(base + j * CH, CH)], idxd[b].at[0],
                         sem_i[b])
        pltpu.async_copy(src_h.at[pl.ds(base + j * CH, CH)], idxs[b].at[0],
                         sem_i[b])

    def iwait(b):
        pltpu.make_async_copy(dst_h.at[pl.ds(base, CH)], idxd[b].at[0],
                              sem_i[b]).wait()
        pltpu.make_async_copy(dst_h.at[pl.ds(base, CH)], idxs[b].at[0],
                              sem_i[b]).wait()

    def gfire(b):
        return pltpu.async_copy(tab_h.at[idxs[b].at[0]], rows_v.at[b],
                                sem_g[b])

    def sfire(b):
        return pltpu.async_copy(rows_v.at[b], acc_sh.at[idxd[b].at[0]],
                                sem_s[b], add=True)

    for b in range(NRING):
        pre(b, b)
    for b in range(NRING):
        iwait(b)

    def body(t, _):
        j = NRING * t
        gd = [gfire(b) for b in range(NRING)]
        sd = []
        for b in range(NRING):
            gd[b].wait()
            sd.append(sfire(b))
        for b in range(NRING):
            sd[b].wait()
            pre(j + NRING + b, b)
        for b in range(NRING):
            iwait(b)
        return 0

    lax.fori_loop(0, NT, body, 0)
    j = NRING * NT
    gd = [gfire(b) for b in range(NRING)]
    sd = []
    for b in range(NRING):
        gd[b].wait()
        sd.append(sfire(b))
    sd[0].wait()
    pre(j + 3, 0)
    iwait(0)
    sd[1].wait()
    pre(j + 4, 1)
    iwait(1)
    g0 = gfire(0)
    g1 = gfire(1)
    sd[2].wait()
    g0.wait()
    s0 = sfire(0)
    g1.wait()
    s1 = sfire(1)
    s0.wait()
    s1.wait()
    plsc.subcore_barrier()
    pltpu.sync_copy(acc_sh.at[pl.ds(s * RPS, RPS)],
                    out_h.at[c, pl.ds(s * RPS, RPS)])


@jax.jit
def _prop_call(src, dst, table, zeros):
    f = functools.partial(
        pl.kernel,
        out_type=jax.ShapeDtypeStruct((NC, N2, D), jnp.float32),
        mesh=_mesh,
        scratch_types=[
            pltpu.VMEM((1, CH), jnp.int32),
            pltpu.VMEM((1, CH), jnp.int32),
            pltpu.VMEM((1, CH), jnp.int32),
            pltpu.VMEM((1, CH), jnp.int32),
            pltpu.VMEM((1, CH), jnp.int32),
            pltpu.VMEM((1, CH), jnp.int32),
            pltpu.VMEM((NRING, CH, D), jnp.float32),
            pltpu.VMEM_SHARED((N2, D), jnp.float32),
        ] + [pltpu.SemaphoreType.DMA] * 9,
    )(_prop_body)
    return f(src, dst, table, zeros)


R_BLK = 2048
N_BLK = N2 // R_BLK


def _dinv_from_degp(degp_blk):
    deg = degp_blk[0, :, 0] + degp_blk[1, :, 0] + 1.0
    return lax.rsqrt(deg)


def _scale_body(x_ref, w_ref, degp_ref, out_ref):
    h = jnp.dot(x_ref[...], w_ref[...], preferred_element_type=jnp.float32)
    dinv = _dinv_from_degp(degp_ref[...])
    out_ref[...] = h * dinv[:, None]


@jax.jit
def _scale_call(x, W1, degP):
    return pl.pallas_call(
        _scale_body,
        grid=(N_BLK,),
        in_specs=[
            pl.BlockSpec((R_BLK, D), lambda i: (i, 0)),
            pl.BlockSpec((D, D), lambda i: (0, 0)),
            pl.BlockSpec((NC, R_BLK, 16), lambda i: (0, i, 0)),
        ],
        out_specs=pl.BlockSpec((R_BLK, D), lambda i: (i, 0)),
        out_shape=jax.ShapeDtypeStruct((N2, D), jnp.float32),
    )(x, W1, degP)


def _relu_body(p_ref, hp_ref, degp_ref, b1_ref, out_ref):
    dinv = _dinv_from_degp(degp_ref[...])
    agg = dinv[:, None] * (p_ref[0] + p_ref[1] + hp_ref[...]) + b1_ref[...]
    out_ref[...] = jnp.maximum(agg, 0.0) * dinv[:, None]


@jax.jit
def _relu_call(P, h1p, degP, b1):
    return pl.pallas_call(
        _relu_body,
        grid=(N_BLK,),
        in_specs=[
            pl.BlockSpec((NC, R_BLK, D), lambda i: (0, i, 0)),
            pl.BlockSpec((R_BLK, D), lambda i: (i, 0)),
            pl.BlockSpec((NC, R_BLK, 16), lambda i: (0, i, 0)),
            pl.BlockSpec((1, D), lambda i: (0, 0)),
        ],
        out_specs=pl.BlockSpec((R_BLK, D), lambda i: (i, 0)),
        out_shape=jax.ShapeDtypeStruct((N2, D), jnp.float32),
    )(P, h1p, degP, b1.reshape(1, D))


def _pool_body(q_ref, hp_ref, degp_ref, batch_ref, w2_ref, b2_ref, out_ref,
               acc_s, cnt_s):
    i = pl.program_id(0)

    @pl.when(i == 0)
    def _():
        acc_s[...] = jnp.zeros((G, D), jnp.float32)
        cnt_s[...] = jnp.zeros((G, D), jnp.float32)

    dinv = _dinv_from_degp(degp_ref[...])
    u = dinv[:, None] * (q_ref[0] + q_ref[1] + hp_ref[...])
    b = batch_ref[0, 0, :]
    onehot_t = (b[None, :] == lax.broadcasted_iota(jnp.int32, (G, R_BLK), 0))
    onehot_t = onehot_t.astype(jnp.float32)
    acc_s[...] += jnp.dot(onehot_t, u, preferred_element_type=jnp.float32)
    cnt_s[...] += jnp.dot(onehot_t, jnp.ones((R_BLK, D), jnp.float32),
                          preferred_element_type=jnp.float32)

    @pl.when(i == N_BLK - 1)
    def _():
        mean = acc_s[...] / jnp.maximum(cnt_s[...], 1.0)
        out_ref[...] = (jnp.dot(mean, w2_ref[...],
                                preferred_element_type=jnp.float32)
                        + b2_ref[...])


@jax.jit
def _pool_call(Q, h2p, degP, batch3, W2, b2):
    return pl.pallas_call(
        _pool_body,
        grid=(N_BLK,),
        in_specs=[
            pl.BlockSpec((NC, R_BLK, D), lambda i: (0, i, 0)),
            pl.BlockSpec((R_BLK, D), lambda i: (i, 0)),
            pl.BlockSpec((NC, R_BLK, 16), lambda i: (0, i, 0)),
            pl.BlockSpec((1, 1, R_BLK), lambda i: (i, 0, 0)),
            pl.BlockSpec((D, D), lambda i: (0, 0)),
            pl.BlockSpec((1, D), lambda i: (0, 0)),
        ],
        out_specs=pl.BlockSpec((G, D), lambda i: (0, 0)),
        out_shape=jax.ShapeDtypeStruct((G, D), jnp.float32),
        scratch_shapes=[
            pltpu.VMEM((G, D), jnp.float32),
            pltpu.VMEM((G, D), jnp.float32),
        ],
    )(Q, h2p, degP, batch3, W2, b2.reshape(1, D))


def kernel(x, edge_index, batch, W1, b1, W2, b2):
    zeros = jnp.zeros((ZR, D), jnp.float32)
    zeros16 = jnp.zeros((RPS, 16), jnp.float32)
    src = edge_index[0]
    dst = edge_index[1]
    x_pad = jnp.pad(x, ((0, N2 - N), (0, 0)))
    batch_pad = jnp.pad(batch, (0, N2 - N), constant_values=G)
    degP = _deg_call(dst, zeros16)
    h1p = _scale_call(x_pad, W1, degP)
    P = _prop_call(src, dst, h1p, zeros)
    h2p = _relu_call(P, h1p, degP, b1)
    Q = _prop_call(src, dst, h2p, zeros)
    batch3 = batch_pad.reshape(N_BLK, 1, R_BLK)
    return _pool_call(Q, h2p, degP, batch3, W2, b2)


# speedup vs baseline: 22.7495x; 1.0027x over previous
import functools

import jax
import jax.numpy as jnp
from jax import lax
from jax.experimental import pallas as pl
from jax.experimental.pallas import tpu as pltpu
from jax.experimental.pallas import tpu_sc as plsc

N = 10000
N2 = 10240
E = 320000
D = 128
G = 64

NC = 2
NS = 16
NW = NC * NS
PER_W = E // NW
CH = 80
NCHUNK = PER_W // CH
SUB = 1
CHG = CH * SUB
NCG = PER_W // CHG
RPS = N2 // NS
ZR = 128

_mesh = plsc.VectorSubcoreMesh(core_axis_name="c", subcore_axis_name="s")


NRING = 3
NT = (NCG - 5) // NRING


def _deg_body(dst_h, zeros_h, out_h, id0, id1, id2, ones_v, acc_sh,
              si0, si1, si2, ss0, ss1, ss2):
    c = lax.axis_index("c")
    s = lax.axis_index("s")
    wid = s * NC + c
    base = wid * PER_W
    sem_i = (si0, si1, si2)
    sem_s = (ss0, ss1, ss2)
    idxd = (id0, id1, id2)

    def fill_ones(i, _):
        ones_v[i, :] = jnp.ones((16,), jnp.float32)
        return 0

    lax.fori_loop(0, CH, fill_ones, 0)
    pltpu.async_copy(zeros_h, acc_sh.at[pl.ds(s * RPS, RPS)], ss0)
    pltpu.make_async_copy(zeros_h, acc_sh.at[pl.ds(s * RPS, RPS)], ss0).wait()
    plsc.subcore_barrier()

    def pre(j, b):
        pltpu.async_copy(dst_h.at[pl.ds(base + j * CH, CH)], idxd[b].at[0],
                         sem_i[b])

    def iwait(b):
        pltpu.make_async_copy(dst_h.at[pl.ds(base, CH)], idxd[b].at[0],
                              sem_i[b]).wait()

    def sfire(b):
        return pltpu.async_copy(ones_v, acc_sh.at[idxd[b].at[0]], sem_s[b],
                                add=True)

    for b in range(NRING):
        pre(b, b)
    for b in range(NRING):
        iwait(b)

    def body(t, _):
        j = NRING * t
        sd = [sfire(b) for b in range(NRING)]
        for b in range(NRING):
            sd[b].wait()
            pre(j + NRING + b, b)
        for b in range(NRING):
            iwait(b)
        return 0

    lax.fori_loop(0, NT, body, 0)
    j = NRING * NT
    sd = [sfire(b) for b in range(NRING)]
    sd[0].wait()
    pre(j + 3, 0)
    iwait(0)
    s0 = sfire(0)
    sd[1].wait()
    pre(j + 4, 1)
    iwait(1)
    s1 = sfire(1)
    sd[2].wait()
    s0.wait()
    s1.wait()
    plsc.subcore_barrier()
    pltpu.sync_copy(acc_sh.at[pl.ds(s * RPS, RPS)],
                    out_h.at[c, pl.ds(s * RPS, RPS)])


@jax.jit
def _deg_call(dst, zeros16):
    f = functools.partial(
        pl.kernel,
        out_type=jax.ShapeDtypeStruct((NC, N2, 16), jnp.float32),
        mesh=_mesh,
        scratch_types=[
            pltpu.VMEM((1, CH), jnp.int32),
            pltpu.VMEM((1, CH), jnp.int32),
            pltpu.VMEM((1, CH), jnp.int32),
            pltpu.VMEM((CH, 16), jnp.float32),
            pltpu.VMEM_SHARED((N2, 16), jnp.float32),
        ] + [pltpu.SemaphoreType.DMA] * 6,
    )(_deg_body)
    return f(dst, zeros16)


def _prop_body(src_h, dst_h, tab_h, zeros_h, out_h, is0, is1, is2,
               id0, id1, id2, rows_v, acc_sh,
               sg0, sg1, sg2, si0, si1, si2, ss0, ss1, ss2):
    c = lax.axis_index("c")
    s = lax.axis_index("s")
    wid = s * NC + c
    base = wid * PER_W
    sem_g = (sg0, sg1, sg2)
    sem_i = (si0, si1, si2)
    sem_s = (ss0, ss1, ss2)
    idxs = (is0, is1, is2)
    idxd = (id0, id1, id2)

    for k in range(RPS // ZR):
        pltpu.async_copy(zeros_h, acc_sh.at[pl.ds(s * RPS + k * ZR, ZR)], ss0)
    for k in range(RPS // ZR):
        pltpu.make_async_copy(zeros_h, acc_sh.at[pl.ds(s * RPS, ZR)],
                              ss0).wait()
    plsc.subcore_barrier()

    def pre(j, b):
        pltpu.async_copy(dst_h.at[pl.ds(base + j * CH, CH)], idxd[b].at[0],
                         sem_i[b])
        pltpu.async_copy(src_h.at[pl.ds(base + j * CH, CH)], idxs[b].at[0],
                         sem_i[b])

    def iwait(b):
        pltpu.make_async_copy(dst_h.at[pl.ds(base, CH)], idxd[b].at[0],
                              sem_i[b]).wait()
        pltpu.make_async_copy(dst_h.at[pl.ds(base, CH)], idxs[b].at[0],
                              sem_i[b]).wait()

    def gfire(b):
        return pltpu.async_copy(tab_h.at[idxs[b].at[0]], rows_v.at[b],
                                sem_g[b])

    def sfire(b):
        return pltpu.async_copy(rows_v.at[b], acc_sh.at[idxd[b].at[0]],
                                sem_s[b], add=True)

    for b in range(NRING):
        pre(b, b)
    for b in range(NRING):
        iwait(b)

    def body(t, _):
        j = NRING * t
        gd = [gfire(b) for b in range(NRING)]
        sd = []
        for b in range(NRING):
            gd[b].wait()
            sd.append(sfire(b))
        for b in range(NRING):
            sd[b].wait()
            pre(j + NRING + b, b)
        for b in range(NRING):
            iwait(b)
        return 0

    lax.fori_loop(0, NT, body, 0)
    j = NRING * NT
    gd = [gfire(b) for b in range(NRING)]
    sd = []
    for b in range(NRING):
        gd[b].wait()
        sd.append(sfire(b))
    sd[0].wait()
    pre(j + 3, 0)
    iwait(0)
    g0 = gfire(0)
    sd[1].wait()
    pre(j + 4, 1)
    iwait(1)
    g1 = gfire(1)
    g0.wait()
    s0 = sfire(0)
    g1.wait()
    s1 = sfire(1)
    sd[2].wait()
    s0.wait()
    s1.wait()
    plsc.subcore_barrier()
    pltpu.sync_copy(acc_sh.at[pl.ds(s * RPS, RPS)],
                    out_h.at[c, pl.ds(s * RPS, RPS)])


@jax.jit
def _prop_call(src, dst, table, zeros):
    f = functools.partial(
        pl.kernel,
        out_type=jax.ShapeDtypeStruct((NC, N2, D), jnp.float32),
        mesh=_mesh,
        scratch_types=(
            [pltpu.VMEM((1, CH), jnp.int32) for _ in range(6)] + [
                pltpu.VMEM((NRING, CH, D), jnp.float32),
                pltpu.VMEM_SHARED((N2, D), jnp.float32),
            ] + [pltpu.SemaphoreType.DMA] * 9
        ),
    )(_prop_body)
    return f(src, dst, table, zeros)


R_BLK = 2048
N_BLK = N2 // R_BLK


def _dinv_from_degp(degp_blk):
    deg = degp_blk[0, :, 0] + degp_blk[1, :, 0] + 1.0
    return lax.rsqrt(deg)


def _scale_body(x_ref, w_ref, degp_ref, out_ref):
    h = jnp.dot(x_ref[...], w_ref[...], preferred_element_type=jnp.float32)
    dinv = _dinv_from_degp(degp_ref[...])
    out_ref[...] = h * dinv[:, None]


@jax.jit
def _scale_call(x, W1, degP):
    return pl.pallas_call(
        _scale_body,
        grid=(N_BLK,),
        in_specs=[
            pl.BlockSpec((R_BLK, D), lambda i: (i, 0)),
            pl.BlockSpec((D, D), lambda i: (0, 0)),
            pl.BlockSpec((NC, R_BLK, 16), lambda i: (0, i, 0)),
        ],
        out_specs=pl.BlockSpec((R_BLK, D), lambda i: (i, 0)),
        out_shape=jax.ShapeDtypeStruct((N2, D), jnp.float32),
    )(x, W1, degP)


def _relu_body(p_ref, hp_ref, degp_ref, b1_ref, out_ref):
    dinv = _dinv_from_degp(degp_ref[...])
    agg = dinv[:, None] * (p_ref[0] + p_ref[1] + hp_ref[...]) + b1_ref[...]
    out_ref[...] = jnp.maximum(agg, 0.0) * dinv[:, None]


@jax.jit
def _relu_call(P, h1p, degP, b1):
    return pl.pallas_call(
        _relu_body,
        grid=(N_BLK,),
        in_specs=[
            pl.BlockSpec((NC, R_BLK, D), lambda i: (0, i, 0)),
            pl.BlockSpec((R_BLK, D), lambda i: (i, 0)),
            pl.BlockSpec((NC, R_BLK, 16), lambda i: (0, i, 0)),
            pl.BlockSpec((1, D), lambda i: (0, 0)),
        ],
        out_specs=pl.BlockSpec((R_BLK, D), lambda i: (i, 0)),
        out_shape=jax.ShapeDtypeStruct((N2, D), jnp.float32),
    )(P, h1p, degP, b1.reshape(1, D))


def _pool_body(q_ref, hp_ref, degp_ref, batch_ref, w2_ref, b2_ref, out_ref,
               acc_s, cnt_s):
    i = pl.program_id(0)

    @pl.when(i == 0)
    def _():
        acc_s[...] = jnp.zeros((G, D), jnp.float32)
        cnt_s[...] = jnp.zeros((G, D), jnp.float32)

    dinv = _dinv_from_degp(degp_ref[...])
    u = dinv[:, None] * (q_ref[0] + q_ref[1] + hp_ref[...])
    b = batch_ref[0, 0, :]
    onehot_t = (b[None, :] == lax.broadcasted_iota(jnp.int32, (G, R_BLK), 0))
    onehot_t = onehot_t.astype(jnp.float32)
    acc_s[...] += jnp.dot(onehot_t, u, preferred_element_type=jnp.float32)
    cnt_s[...] += jnp.dot(onehot_t, jnp.ones((R_BLK, D), jnp.float32),
                          preferred_element_type=jnp.float32)

    @pl.when(i == N_BLK - 1)
    def _():
        mean = acc_s[...] / jnp.maximum(cnt_s[...], 1.0)
        out_ref[...] = (jnp.dot(mean, w2_ref[...],
                                preferred_element_type=jnp.float32)
                        + b2_ref[...])


@jax.jit
def _pool_call(Q, h2p, degP, batch3, W2, b2):
    return pl.pallas_call(
        _pool_body,
        grid=(N_BLK,),
        in_specs=[
            pl.BlockSpec((NC, R_BLK, D), lambda i: (0, i, 0)),
            pl.BlockSpec((R_BLK, D), lambda i: (i, 0)),
            pl.BlockSpec((NC, R_BLK, 16), lambda i: (0, i, 0)),
            pl.BlockSpec((1, 1, R_BLK), lambda i: (i, 0, 0)),
            pl.BlockSpec((D, D), lambda i: (0, 0)),
            pl.BlockSpec((1, D), lambda i: (0, 0)),
        ],
        out_specs=pl.BlockSpec((G, D), lambda i: (0, 0)),
        out_shape=jax.ShapeDtypeStruct((G, D), jnp.float32),
        scratch_shapes=[
            pltpu.VMEM((G, D), jnp.float32),
            pltpu.VMEM((G, D), jnp.float32),
        ],
    )(Q, h2p, degP, batch3, W2, b2.reshape(1, D))


def kernel(x, edge_index, batch, W1, b1, W2, b2):
    zeros = jnp.zeros((ZR, D), jnp.float32)
    zeros16 = jnp.zeros((RPS, 16), jnp.float32)
    src = edge_index[0]
    dst = edge_index[1]
    x_pad = jnp.pad(x, ((0, N2 - N), (0, 0)))
    batch_pad = jnp.pad(batch, (0, N2 - N), constant_values=G)
    degP = _deg_call(dst, zeros16)
    h1p = _scale_call(x_pad, W1, degP)
    P = _prop_call(src, dst, h1p, zeros)
    h2p = _relu_call(P, h1p, degP, b1)
    Q = _prop_call(src, dst, h2p, zeros)
    batch3 = batch_pad.reshape(N_BLK, 1, R_BLK)
    return _pool_call(Q, h2p, degP, batch3, W2, b2)
